# Initial kernel scaffold; baseline (speedup 1.0000x reference)
import jax
import jax.numpy as jnp
from jax.experimental import pallas as pl


def kernel(moves_mem, idx, val):
    raise NotImplementedError("write your pallas kernel here")


# speedup vs baseline: 1.5776x; 1.5776x over previous
import functools

import jax
import jax.numpy as jnp
from jax import lax
from jax.experimental import pallas as pl
from jax.experimental.pallas import tpu as pltpu
from jax.experimental.pallas import tpu_sc as plsc


def _probs_body(idx_ref, val_ref, rank_ref, out_ref):
    idx = idx_ref[...]
    val = val_ref[...]
    rk = rank_ref[...]
    R, K = idx.shape
    eq = idx[:, :, None] == idx[:, None, :]
    best = jnp.max(jnp.where(eq, rk[:, None, :], -1), axis=2)
    winner = best == rk
    onehot = eq & (rk[:, None, :] == best[:, :, None])
    win_val = jnp.sum(jnp.where(onehot, val[:, None, :], 0.0), axis=2)
    m = jnp.max(jnp.where(winner, val, -1e30), axis=1, keepdims=True)
    denom = jnp.sum(jnp.where(winner, jnp.exp(val - m), 0.0), axis=1,
                    keepdims=True)
    out_ref[...] = jnp.exp(win_val - m) / denom


def _compute_probs(idx, val, rank, block_rows=128):
    B, K = idx.shape
    spec = pl.BlockSpec((block_rows, K), lambda i: (i, 0))
    return pl.pallas_call(
        _probs_body,
        grid=(B // block_rows,),
        in_specs=[spec, spec, spec],
        out_specs=spec,
        out_shape=jax.ShapeDtypeStruct((B, K), jnp.float32),
    )(idx, val, rank)


@functools.lru_cache(maxsize=None)
def _make_scatter(B, D, K):
    info = plsc.get_sparse_core_info()
    NC, NS = info.num_cores, info.num_subcores
    NW = NC * NS
    RPW = B // NW
    RPT = 8
    NG = RPW // RPT
    NCHUNK = K // 16
    mesh = plsc.VectorSubcoreMesh(core_axis_name="c", subcore_axis_name="s")

    @functools.partial(
        pl.kernel,
        mesh=mesh,
        compiler_params=pltpu.CompilerParams(needs_layout_passes=False),
        out_type=jax.ShapeDtypeStruct((B * D,), jnp.float32),
        scratch_types=[
            pltpu.VMEM((RPW * K,), jnp.int32),
            pltpu.VMEM((RPW * K,), jnp.float32),
            pltpu.VMEM((RPT * D,), jnp.float32),
        ],
    )
    def sc(idx_hbm, prob_hbm, z_hbm, out_hbm, idxb, prb, buf):
        wid = lax.axis_index("s") * NC + lax.axis_index("c")
        base = wid * RPW
        pltpu.sync_copy(idx_hbm.at[pl.ds(base * K, RPW * K)], idxb)
        pltpu.sync_copy(prob_hbm.at[pl.ds(base * K, RPW * K)], prb)
        pltpu.sync_copy(z_hbm, buf)

        zv = jnp.zeros((16,), jnp.float32)

        def group(g, carry):
            def scatter_row(i, carry2):
                roff = i * D
                koff = (g * RPT + i) * K
                for c in range(NCHUNK):
                    iv = idxb[pl.ds(koff + c * 16, 16)]
                    pv = prb[pl.ds(koff + c * 16, 16)]
                    plsc.store_scatter(buf, [iv + roff], pv)
                return carry2

            lax.fori_loop(0, RPT, scatter_row, 0)
            pltpu.sync_copy(
                buf, out_hbm.at[pl.ds((base + g * RPT) * D, RPT * D)])

            def zero_row(i, carry2):
                roff = i * D
                koff = (g * RPT + i) * K
                for c in range(NCHUNK):
                    iv = idxb[pl.ds(koff + c * 16, 16)]
                    plsc.store_scatter(buf, [iv + roff], zv)
                return carry2

            lax.fori_loop(0, RPT, zero_row, 0)
            return carry

        lax.fori_loop(0, NG, group, 0)

    return sc, RPT


def kernel(moves_mem, idx, val):
    B, D = moves_mem.shape
    K = idx.shape[1]
    keys = (idx * B + jnp.arange(B, dtype=jnp.int32)[:, None]).reshape(-1)
    pos = jnp.arange(B * K, dtype=jnp.float32)
    _, sp = lax.sort((keys, pos), dimension=0, is_stable=False, num_keys=1)
    rank = (
        jnp.zeros((B * K,), jnp.int32)
        .at[sp.astype(jnp.int32)]
        .set(jnp.arange(B * K, dtype=jnp.int32), unique_indices=True)
        .reshape(B, K)
    )
    probs = _compute_probs(idx, val, rank)
    sc, rpt = _make_scatter(B, D, K)
    zeros = jnp.zeros((rpt * D,), jnp.float32)
    out1d = sc(idx.reshape(-1), probs.reshape(-1), zeros)
    return out1d.reshape(B, D)


# speedup vs baseline: 2.6442x; 1.6761x over previous
import functools

import jax
import jax.numpy as jnp
from jax import lax
from jax.experimental import pallas as pl
from jax.experimental.pallas import tpu as pltpu
from jax.experimental.pallas import tpu_sc as plsc


def _probs_body(idx_ref, val_ref, rank_ref, out_ref):
    idx = idx_ref[...]
    val = val_ref[...]
    rk = rank_ref[...]
    R, K = idx.shape
    eq = idx[:, :, None] == idx[:, None, :]
    best = jnp.max(jnp.where(eq, rk[:, None, :], -1), axis=2)
    winner = best == rk
    onehot = eq & (rk[:, None, :] == best[:, :, None])
    win_val = jnp.sum(jnp.where(onehot, val[:, None, :], 0.0), axis=2)
    m = jnp.max(jnp.where(winner, val, -1e30), axis=1, keepdims=True)
    denom = jnp.sum(jnp.where(winner, jnp.exp(val - m), 0.0), axis=1,
                    keepdims=True)
    out_ref[...] = jnp.exp(win_val - m) / denom


def _compute_probs(idx, val, rank, block_rows=128):
    B, K = idx.shape
    spec = pl.BlockSpec((block_rows, K), lambda i: (i, 0))
    return pl.pallas_call(
        _probs_body,
        grid=(B // block_rows,),
        in_specs=[spec, spec, spec],
        out_specs=spec,
        out_shape=jax.ShapeDtypeStruct((B, K), jnp.float32),
    )(idx, val, rank)


@functools.lru_cache(maxsize=None)
def _make_rank_scatter(BK):
    info = plsc.get_sparse_core_info()
    NC, NS = info.num_cores, info.num_subcores
    NW = NC * NS
    CH = BK // NW
    NR = CH // 128
    mesh = plsc.VectorSubcoreMesh(core_axis_name="c", subcore_axis_name="s")

    @functools.partial(
        pl.kernel,
        mesh=mesh,
        compiler_params=pltpu.CompilerParams(needs_layout_passes=False),
        out_type=jax.ShapeDtypeStruct((BK,), jnp.int32),
        scratch_types=[
            pltpu.VMEM((NR, 128), jnp.int32),
            pltpu.VMEM((NR, 128), jnp.int32),
            pltpu.SemaphoreType.DMA,
        ],
    )
    def rk(sp_hbm, iota_hbm, rank_hbm, idxv, valv, sem):
        wid = lax.axis_index("s") * NC + lax.axis_index("c")
        base = wid * NR
        pltpu.sync_copy(sp_hbm.at[pl.ds(base, NR)], idxv)
        pltpu.sync_copy(iota_hbm.at[pl.ds(base, NR)], valv)
        handles = []
        for j in range(NR):
            handles.append(
                pltpu.async_copy(valv.at[j], rank_hbm.at[idxv.at[j]], sem))
        for h in handles:
            h.wait()

    return rk


@functools.lru_cache(maxsize=None)
def _make_scatter(B, D, K):
    info = plsc.get_sparse_core_info()
    NC, NS = info.num_cores, info.num_subcores
    NW = NC * NS
    RPW = B // NW
    RPT = 8
    NG = RPW // RPT
    NCHUNK = K // 16
    mesh = plsc.VectorSubcoreMesh(core_axis_name="c", subcore_axis_name="s")

    @functools.partial(
        pl.kernel,
        mesh=mesh,
        compiler_params=pltpu.CompilerParams(needs_layout_passes=False),
        out_type=jax.ShapeDtypeStruct((B * D,), jnp.float32),
        scratch_types=[
            pltpu.VMEM((RPW * K,), jnp.int32),
            pltpu.VMEM((RPW * K,), jnp.float32),
            pltpu.VMEM((RPT * D,), jnp.float32),
        ],
    )
    def sc(idx_hbm, prob_hbm, z_hbm, out_hbm, idxb, prb, buf):
        wid = lax.axis_index("s") * NC + lax.axis_index("c")
        base = wid * RPW
        pltpu.sync_copy(idx_hbm.at[pl.ds(base * K, RPW * K)], idxb)
        pltpu.sync_copy(prob_hbm.at[pl.ds(base * K, RPW * K)], prb)
        pltpu.sync_copy(z_hbm, buf)

        zv = jnp.zeros((16,), jnp.float32)

        def group(g, carry):
            def scatter_row(i, carry2):
                roff = i * D
                koff = (g * RPT + i) * K
                for c in range(NCHUNK):
                    iv = idxb[pl.ds(koff + c * 16, 16)]
                    pv = prb[pl.ds(koff + c * 16, 16)]
                    plsc.store_scatter(buf, [iv + roff], pv)
                return carry2

            lax.fori_loop(0, RPT, scatter_row, 0)
            pltpu.sync_copy(
                buf, out_hbm.at[pl.ds((base + g * RPT) * D, RPT * D)])

            def zero_row(i, carry2):
                roff = i * D
                koff = (g * RPT + i) * K
                for c in range(NCHUNK):
                    iv = idxb[pl.ds(koff + c * 16, 16)]
                    plsc.store_scatter(buf, [iv + roff], zv)
                return carry2

            lax.fori_loop(0, RPT, zero_row, 0)
            return carry

        lax.fori_loop(0, NG, group, 0)

    return sc, RPT


def kernel(moves_mem, idx, val):
    B, D = moves_mem.shape
    K = idx.shape[1]
    keys = (idx * B + jnp.arange(B, dtype=jnp.int32)[:, None]).reshape(-1)
    pos = jnp.arange(B * K, dtype=jnp.float32)
    _, sp = lax.sort((keys, pos), dimension=0, is_stable=False, num_keys=1)
    rk, _ = _make_rank_scatter(B * K), None
    rank = rk(
        sp.astype(jnp.int32).reshape(-1, 128),
        jnp.arange(B * K, dtype=jnp.int32).reshape(-1, 128),
    ).reshape(B, K)
    probs = _compute_probs(idx, val, rank)
    sc, rpt = _make_scatter(B, D, K)
    zeros = jnp.zeros((rpt * D,), jnp.float32)
    out1d = sc(idx.reshape(-1), probs.reshape(-1), zeros)
    return out1d.reshape(B, D)


# speedup vs baseline: 3.6053x; 1.3634x over previous
import functools

import jax
import jax.numpy as jnp
from jax import lax
from jax.experimental import pallas as pl
from jax.experimental.pallas import tpu as pltpu
from jax.experimental.pallas import tpu_sc as plsc


def _probs_body(idx_ref, val_ref, win_ref, out_ref):
    idx = idx_ref[...]
    val = val_ref[...]
    win = win_ref[...]
    eq = idx[:, :, None] == idx[:, None, :]
    sel = eq & (win[:, None, :] == 1)
    win_val = jnp.max(jnp.where(sel, val[:, None, :], -1e30), axis=2)
    winner = win == 1
    m = jnp.max(jnp.where(winner, val, -1e30), axis=1, keepdims=True)
    denom = jnp.sum(jnp.where(winner, jnp.exp(val - m), 0.0), axis=1,
                    keepdims=True)
    out_ref[...] = jnp.exp(win_val - m) / denom


def _compute_probs(idx, val, rank, block_rows=128):
    B, K = idx.shape
    spec = pl.BlockSpec((block_rows, K), lambda i: (i, 0))
    return pl.pallas_call(
        _probs_body,
        grid=(B // block_rows,),
        in_specs=[spec, spec, spec],
        out_specs=spec,
        out_shape=jax.ShapeDtypeStruct((B, K), jnp.float32),
    )(idx, val, rank)


@functools.lru_cache(maxsize=None)
def _make_rank_scatter(BK):
    info = plsc.get_sparse_core_info()
    NC, NS = info.num_cores, info.num_subcores
    NW = NC * NS
    CH = BK // NW
    NR = CH // 128
    mesh = plsc.VectorSubcoreMesh(core_axis_name="c", subcore_axis_name="s")

    @functools.partial(
        pl.kernel,
        mesh=mesh,
        compiler_params=pltpu.CompilerParams(needs_layout_passes=False),
        out_type=jax.ShapeDtypeStruct((BK,), jnp.int32),
        scratch_types=[
            pltpu.VMEM((NR, 128), jnp.int32),
            pltpu.VMEM((NR, 128), jnp.int32),
            pltpu.SemaphoreType.DMA,
        ],
    )
    def rk(sp_hbm, iota_hbm, rank_hbm, idxv, valv, sem):
        wid = lax.axis_index("s") * NC + lax.axis_index("c")
        base = wid * NR
        pltpu.sync_copy(sp_hbm.at[pl.ds(base, NR)], idxv)
        pltpu.sync_copy(iota_hbm.at[pl.ds(base, NR)], valv)
        handles = []
        for j in range(NR):
            handles.append(
                pltpu.async_copy(valv.at[j], rank_hbm.at[idxv.at[j]], sem))
        for h in handles:
            h.wait()

    return rk


@functools.lru_cache(maxsize=None)
def _make_scatter(B, D, K):
    info = plsc.get_sparse_core_info()
    NC, NS = info.num_cores, info.num_subcores
    NW = NC * NS
    RPW = B // NW
    RPT = 8
    NG = RPW // RPT
    NCHUNK = K // 16
    mesh = plsc.VectorSubcoreMesh(core_axis_name="c", subcore_axis_name="s")

    @functools.partial(
        pl.kernel,
        mesh=mesh,
        compiler_params=pltpu.CompilerParams(needs_layout_passes=False),
        out_type=jax.ShapeDtypeStruct((B * D,), jnp.float32),
        scratch_types=[
            pltpu.VMEM((RPW * K,), jnp.int32),
            pltpu.VMEM((RPW * K,), jnp.float32),
            pltpu.VMEM((RPT * D,), jnp.float32),
        ],
    )
    def sc(idx_hbm, prob_hbm, z_hbm, out_hbm, idxb, prb, buf):
        wid = lax.axis_index("s") * NC + lax.axis_index("c")
        base = wid * RPW
        pltpu.sync_copy(idx_hbm.at[pl.ds(base * K, RPW * K)], idxb)
        pltpu.sync_copy(prob_hbm.at[pl.ds(base * K, RPW * K)], prb)
        pltpu.sync_copy(z_hbm, buf)

        zv = jnp.zeros((16,), jnp.float32)

        def group(g, carry):
            def scatter_row(i, carry2):
                roff = i * D
                koff = (g * RPT + i) * K
                for c in range(NCHUNK):
                    iv = idxb[pl.ds(koff + c * 16, 16)]
                    pv = prb[pl.ds(koff + c * 16, 16)]
                    plsc.store_scatter(buf, [iv + roff], pv)
                return carry2

            lax.fori_loop(0, RPT, scatter_row, 0)
            pltpu.sync_copy(
                buf, out_hbm.at[pl.ds((base + g * RPT) * D, RPT * D)])

            def zero_row(i, carry2):
                roff = i * D
                koff = (g * RPT + i) * K
                for c in range(NCHUNK):
                    iv = idxb[pl.ds(koff + c * 16, 16)]
                    plsc.store_scatter(buf, [iv + roff], zv)
                return carry2

            lax.fori_loop(0, RPT, zero_row, 0)
            return carry

        lax.fori_loop(0, NG, group, 0)

    return sc, RPT


def kernel(moves_mem, idx, val):
    B, D = moves_mem.shape
    K = idx.shape[1]
    keys = (idx * B + jnp.arange(B, dtype=jnp.int32)[:, None]).reshape(-1)
    pos = jnp.arange(B * K, dtype=jnp.float32)
    sk, sp = lax.sort((keys, pos), dimension=0, is_stable=False, num_keys=1)
    flag = jnp.concatenate(
        [(sk[:-1] != sk[1:]).astype(jnp.int32),
         jnp.ones((1,), jnp.int32)])
    rk = _make_rank_scatter(B * K)
    rank = rk(
        sp.astype(jnp.int32).reshape(-1, 128),
        flag.reshape(-1, 128),
    ).reshape(B, K)
    probs = _compute_probs(idx, val, rank)
    sc, rpt = _make_scatter(B, D, K)
    zeros = jnp.zeros((rpt * D,), jnp.float32)
    out1d = sc(idx.reshape(-1), probs.reshape(-1), zeros)
    return out1d.reshape(B, D)


# speedup vs baseline: 3.6303x; 1.0069x over previous
import functools

import jax
import jax.numpy as jnp
from jax import lax
from jax.experimental import pallas as pl
from jax.experimental.pallas import tpu as pltpu
from jax.experimental.pallas import tpu_sc as plsc


def _probs_body(idx_ref, val_ref, win_ref, out_ref):
    idx = idx_ref[...]
    val = val_ref[...]
    win = win_ref[...]
    eq = idx[:, :, None] == idx[:, None, :]
    sel = eq & (win[:, None, :] == 1)
    win_val = jnp.max(jnp.where(sel, val[:, None, :], -1e30), axis=2)
    winner = win == 1
    m = jnp.max(jnp.where(winner, val, -1e30), axis=1, keepdims=True)
    denom = jnp.sum(jnp.where(winner, jnp.exp(val - m), 0.0), axis=1,
                    keepdims=True)
    out_ref[...] = jnp.exp(win_val - m) / denom


def _compute_probs(idx, val, rank, block_rows=512):
    B, K = idx.shape
    spec = pl.BlockSpec((block_rows, K), lambda i: (i, 0))
    return pl.pallas_call(
        _probs_body,
        grid=(B // block_rows,),
        in_specs=[spec, spec, spec],
        out_specs=spec,
        out_shape=jax.ShapeDtypeStruct((B, K), jnp.float32),
    )(idx, val, rank)


@functools.lru_cache(maxsize=None)
def _make_rank_scatter(BK):
    info = plsc.get_sparse_core_info()
    NC, NS = info.num_cores, info.num_subcores
    NW = NC * NS
    CH = BK // NW
    NR = CH // 128
    mesh = plsc.VectorSubcoreMesh(core_axis_name="c", subcore_axis_name="s")

    @functools.partial(
        pl.kernel,
        mesh=mesh,
        compiler_params=pltpu.CompilerParams(needs_layout_passes=False),
        out_type=jax.ShapeDtypeStruct((BK,), jnp.int32),
        scratch_types=[
            pltpu.VMEM((NR, 128), jnp.int32),
            pltpu.VMEM((NR, 128), jnp.int32),
            pltpu.SemaphoreType.DMA,
        ],
    )
    def rk(sp_hbm, iota_hbm, rank_hbm, idxv, valv, sem):
        wid = lax.axis_index("s") * NC + lax.axis_index("c")
        base = wid * NR
        pltpu.sync_copy(sp_hbm.at[pl.ds(base, NR)], idxv)
        pltpu.sync_copy(iota_hbm.at[pl.ds(base, NR)], valv)
        handles = []
        for j in range(NR):
            handles.append(
                pltpu.async_copy(valv.at[j], rank_hbm.at[idxv.at[j]], sem))
        for h in handles:
            h.wait()

    return rk


@functools.lru_cache(maxsize=None)
def _make_scatter(B, D, K):
    info = plsc.get_sparse_core_info()
    NC, NS = info.num_cores, info.num_subcores
    NW = NC * NS
    RPW = B // NW
    RPT = 8
    NG = RPW // RPT
    NCHUNK = K // 16
    mesh = plsc.VectorSubcoreMesh(core_axis_name="c", subcore_axis_name="s")

    @functools.partial(
        pl.kernel,
        mesh=mesh,
        compiler_params=pltpu.CompilerParams(needs_layout_passes=False),
        out_type=jax.ShapeDtypeStruct((B * D,), jnp.float32),
        scratch_types=[
            pltpu.VMEM((RPW * K,), jnp.int32),
            pltpu.VMEM((RPW * K,), jnp.float32),
            pltpu.VMEM((RPT * D,), jnp.float32),
        ],
    )
    def sc(idx_hbm, prob_hbm, z_hbm, out_hbm, idxb, prb, buf):
        wid = lax.axis_index("s") * NC + lax.axis_index("c")
        base = wid * RPW
        pltpu.sync_copy(idx_hbm.at[pl.ds(base * K, RPW * K)], idxb)
        pltpu.sync_copy(prob_hbm.at[pl.ds(base * K, RPW * K)], prb)
        pltpu.sync_copy(z_hbm, buf)

        zv = jnp.zeros((16,), jnp.float32)

        def group(g, carry):
            def scatter_row(i, carry2):
                roff = i * D
                koff = (g * RPT + i) * K
                for c in range(NCHUNK):
                    iv = idxb[pl.ds(koff + c * 16, 16)]
                    pv = prb[pl.ds(koff + c * 16, 16)]
                    plsc.store_scatter(buf, [iv + roff], pv)
                return carry2

            lax.fori_loop(0, RPT, scatter_row, 0)
            pltpu.sync_copy(
                buf, out_hbm.at[pl.ds((base + g * RPT) * D, RPT * D)])

            def zero_row(i, carry2):
                roff = i * D
                koff = (g * RPT + i) * K
                for c in range(NCHUNK):
                    iv = idxb[pl.ds(koff + c * 16, 16)]
                    plsc.store_scatter(buf, [iv + roff], zv)
                return carry2

            lax.fori_loop(0, RPT, zero_row, 0)
            return carry

        lax.fori_loop(0, NG, group, 0)

    return sc, RPT


def kernel(moves_mem, idx, val):
    B, D = moves_mem.shape
    K = idx.shape[1]
    keys = (idx * B + jnp.arange(B, dtype=jnp.int32)[:, None]).reshape(-1)
    pos = jnp.arange(B * K, dtype=jnp.float32)
    sk, sp = lax.sort((keys, pos), dimension=0, is_stable=False, num_keys=1)
    flag = jnp.concatenate(
        [(sk[:-1] != sk[1:]).astype(jnp.int32),
         jnp.ones((1,), jnp.int32)])
    rk = _make_rank_scatter(B * K)
    rank = rk(
        sp.astype(jnp.int32).reshape(-1, 128),
        flag.reshape(-1, 128),
    ).reshape(B, K)
    probs = _compute_probs(idx, val, rank)
    sc, rpt = _make_scatter(B, D, K)
    zeros = jnp.zeros((rpt * D,), jnp.float32)
    out1d = sc(idx.reshape(-1), probs.reshape(-1), zeros)
    return out1d.reshape(B, D)


# speedup vs baseline: 4.0383x; 1.1124x over previous
import functools

import jax
import jax.numpy as jnp
from jax import lax
from jax.experimental import pallas as pl
from jax.experimental.pallas import tpu as pltpu
from jax.experimental.pallas import tpu_sc as plsc


def _probs_body(idx_ref, val_ref, win_ref, out_ref):
    idx = idx_ref[...]
    val = val_ref[...]
    win = win_ref[...]
    eq = idx[:, :, None] == idx[:, None, :]
    sel = eq & (win[:, None, :] == 1)
    win_val = jnp.max(jnp.where(sel, val[:, None, :], -1e30), axis=2)
    winner = win == 1
    m = jnp.max(jnp.where(winner, val, -1e30), axis=1, keepdims=True)
    denom = jnp.sum(jnp.where(winner, jnp.exp(val - m), 0.0), axis=1,
                    keepdims=True)
    out_ref[...] = jnp.exp(win_val - m) / denom


def _compute_probs(idx, val, rank, block_rows=512):
    B, K = idx.shape
    spec = pl.BlockSpec((block_rows, K), lambda i: (i, 0))
    return pl.pallas_call(
        _probs_body,
        grid=(B // block_rows,),
        in_specs=[spec, spec, spec],
        out_specs=spec,
        out_shape=jax.ShapeDtypeStruct((B, K), jnp.float32),
    )(idx, val, rank)


@functools.lru_cache(maxsize=None)
def _make_rank_scatter(BK):
    info = plsc.get_sparse_core_info()
    NC, NS = info.num_cores, info.num_subcores
    NW = NC * NS
    CH = BK // NW
    CHUNK = 8192
    NCK = BK // CHUNK
    mesh = plsc.VectorSubcoreMesh(core_axis_name="c", subcore_axis_name="s")

    @functools.partial(
        pl.kernel,
        mesh=mesh,
        compiler_params=pltpu.CompilerParams(needs_layout_passes=False),
        out_type=jax.ShapeDtypeStruct((BK,), jnp.int32),
        scratch_types=[
            pltpu.VMEM((CHUNK,), jnp.int32),
            pltpu.VMEM((CHUNK,), jnp.int32),
            pltpu.VMEM((CH,), jnp.int32),
        ],
    )
    def rk(sp_hbm, val_hbm, rank_hbm, spv, flv, outv):
        wid = lax.axis_index("s") * NC + lax.axis_index("c")
        base = wid * CH

        def chunk(c, carry):
            pltpu.sync_copy(sp_hbm.at[pl.ds(c * CHUNK, CHUNK)], spv)
            pltpu.sync_copy(val_hbm.at[pl.ds(c * CHUNK, CHUNK)], flv)

            def vec(i, carry2):
                iv = spv[pl.ds(i * 16, 16)]
                fv = flv[pl.ds(i * 16, 16)]
                local = iv - base
                mask = (local >= 0) & (local < CH)
                safe = jnp.where(mask, local, 0)
                plsc.store_scatter(outv, [safe], fv, mask=mask)
                return carry2

            lax.fori_loop(0, CHUNK // 16, vec, 0)
            return carry

        lax.fori_loop(0, NCK, chunk, 0)
        pltpu.sync_copy(outv, rank_hbm.at[pl.ds(base, CH)])

    return rk


@functools.lru_cache(maxsize=None)
def _make_scatter(B, D, K):
    info = plsc.get_sparse_core_info()
    NC, NS = info.num_cores, info.num_subcores
    NW = NC * NS
    RPW = B // NW
    RPT = 8
    NG = RPW // RPT
    NCHUNK = K // 16
    mesh = plsc.VectorSubcoreMesh(core_axis_name="c", subcore_axis_name="s")

    @functools.partial(
        pl.kernel,
        mesh=mesh,
        compiler_params=pltpu.CompilerParams(needs_layout_passes=False),
        out_type=jax.ShapeDtypeStruct((B * D,), jnp.float32),
        scratch_types=[
            pltpu.VMEM((RPW * K,), jnp.int32),
            pltpu.VMEM((RPW * K,), jnp.float32),
            pltpu.VMEM((RPT * D,), jnp.float32),
        ],
    )
    def sc(idx_hbm, prob_hbm, z_hbm, out_hbm, idxb, prb, buf):
        wid = lax.axis_index("s") * NC + lax.axis_index("c")
        base = wid * RPW
        pltpu.sync_copy(idx_hbm.at[pl.ds(base * K, RPW * K)], idxb)
        pltpu.sync_copy(prob_hbm.at[pl.ds(base * K, RPW * K)], prb)
        pltpu.sync_copy(z_hbm, buf)

        zv = jnp.zeros((16,), jnp.float32)

        def group(g, carry):
            def scatter_row(i, carry2):
                roff = i * D
                koff = (g * RPT + i) * K
                for c in range(NCHUNK):
                    iv = idxb[pl.ds(koff + c * 16, 16)]
                    pv = prb[pl.ds(koff + c * 16, 16)]
                    plsc.store_scatter(buf, [iv + roff], pv)
                return carry2

            lax.fori_loop(0, RPT, scatter_row, 0)
            pltpu.sync_copy(
                buf, out_hbm.at[pl.ds((base + g * RPT) * D, RPT * D)])

            def zero_row(i, carry2):
                roff = i * D
                koff = (g * RPT + i) * K
                for c in range(NCHUNK):
                    iv = idxb[pl.ds(koff + c * 16, 16)]
                    plsc.store_scatter(buf, [iv + roff], zv)
                return carry2

            lax.fori_loop(0, RPT, zero_row, 0)
            return carry

        lax.fori_loop(0, NG, group, 0)

    return sc, RPT


def kernel(moves_mem, idx, val):
    B, D = moves_mem.shape
    K = idx.shape[1]
    keys = (idx * B + jnp.arange(B, dtype=jnp.int32)[:, None]).reshape(-1)
    pos = jnp.arange(B * K, dtype=jnp.float32)
    sk, sp = lax.sort((keys, pos), dimension=0, is_stable=False, num_keys=1)
    flag = jnp.concatenate(
        [(sk[:-1] != sk[1:]).astype(jnp.int32),
         jnp.ones((1,), jnp.int32)])
    rk = _make_rank_scatter(B * K)
    rank = rk(sp.astype(jnp.int32), flag).reshape(B, K)
    probs = _compute_probs(idx, val, rank)
    sc, rpt = _make_scatter(B, D, K)
    zeros = jnp.zeros((rpt * D,), jnp.float32)
    out1d = sc(idx.reshape(-1), probs.reshape(-1), zeros)
    return out1d.reshape(B, D)


# speedup vs baseline: 4.1091x; 1.0175x over previous
import functools

import jax
import jax.numpy as jnp
from jax import lax
from jax.experimental import pallas as pl
from jax.experimental.pallas import tpu as pltpu
from jax.experimental.pallas import tpu_sc as plsc


def _probs_body(idx_ref, val_ref, win_ref, out_ref):
    idx = idx_ref[...]
    val = val_ref[...]
    win = win_ref[...]
    eq = idx[:, :, None] == idx[:, None, :]
    sel = eq & (win[:, None, :] == 1)
    win_val = jnp.max(jnp.where(sel, val[:, None, :], -1e30), axis=2)
    winner = win == 1
    m = jnp.max(jnp.where(winner, val, -1e30), axis=1, keepdims=True)
    denom = jnp.sum(jnp.where(winner, jnp.exp(val - m), 0.0), axis=1,
                    keepdims=True)
    out_ref[...] = jnp.exp(win_val - m) / denom


def _compute_probs(idx, val, rank, block_rows=512):
    B, K = idx.shape
    spec = pl.BlockSpec((block_rows, K), lambda i: (i, 0))
    return pl.pallas_call(
        _probs_body,
        grid=(B // block_rows,),
        in_specs=[spec, spec, spec],
        out_specs=spec,
        out_shape=jax.ShapeDtypeStruct((B, K), jnp.float32),
    )(idx, val, rank)


@functools.lru_cache(maxsize=None)
def _make_rank_scatter(BK):
    info = plsc.get_sparse_core_info()
    NC, NS = info.num_cores, info.num_subcores
    NW = NC * NS
    CH = BK // NW
    CHUNK = 8192
    NCK = BK // CHUNK
    mesh = plsc.VectorSubcoreMesh(core_axis_name="c", subcore_axis_name="s")

    @functools.partial(
        pl.kernel,
        mesh=mesh,
        compiler_params=pltpu.CompilerParams(needs_layout_passes=False),
        out_type=jax.ShapeDtypeStruct((BK,), jnp.int32),
        scratch_types=[
            pltpu.VMEM((CHUNK,), jnp.int32),
            pltpu.VMEM((CHUNK,), jnp.int32),
            pltpu.VMEM((CH,), jnp.int32),
        ],
    )
    def rk(sp_hbm, val_hbm, rank_hbm, spv, flv, outv):
        wid = lax.axis_index("s") * NC + lax.axis_index("c")
        base = wid * CH

        def chunk(c, carry):
            pltpu.sync_copy(sp_hbm.at[pl.ds(c * CHUNK, CHUNK)], spv)
            pltpu.sync_copy(val_hbm.at[pl.ds(c * CHUNK, CHUNK)], flv)

            def vec(i, carry2):
                for u in range(4):
                    off = (i * 4 + u) * 16
                    iv = spv[pl.ds(off, 16)]
                    fv = flv[pl.ds(off, 16)]
                    local = iv - base
                    mask = (local >= 0) & (local < CH)
                    safe = jnp.where(mask, local, 0)
                    plsc.store_scatter(outv, [safe], fv, mask=mask)
                return carry2

            lax.fori_loop(0, CHUNK // 64, vec, 0)
            return carry

        lax.fori_loop(0, NCK, chunk, 0)
        pltpu.sync_copy(outv, rank_hbm.at[pl.ds(base, CH)])

    return rk


@functools.lru_cache(maxsize=None)
def _make_scatter(B, D, K):
    info = plsc.get_sparse_core_info()
    NC, NS = info.num_cores, info.num_subcores
    NW = NC * NS
    RPW = B // NW
    RPT = 8
    NG = RPW // RPT
    NCHUNK = K // 16
    mesh = plsc.VectorSubcoreMesh(core_axis_name="c", subcore_axis_name="s")

    @functools.partial(
        pl.kernel,
        mesh=mesh,
        compiler_params=pltpu.CompilerParams(needs_layout_passes=False),
        out_type=jax.ShapeDtypeStruct((B * D,), jnp.float32),
        scratch_types=[
            pltpu.VMEM((RPW * K,), jnp.int32),
            pltpu.VMEM((RPW * K,), jnp.float32),
            pltpu.VMEM((RPT * D,), jnp.float32),
        ],
    )
    def sc(idx_hbm, prob_hbm, z_hbm, out_hbm, idxb, prb, buf):
        wid = lax.axis_index("s") * NC + lax.axis_index("c")
        base = wid * RPW
        pltpu.sync_copy(idx_hbm.at[pl.ds(base * K, RPW * K)], idxb)
        pltpu.sync_copy(prob_hbm.at[pl.ds(base * K, RPW * K)], prb)
        pltpu.sync_copy(z_hbm, buf)

        zv = jnp.zeros((16,), jnp.float32)

        def group(g, carry):
            def scatter_row(i, carry2):
                roff = i * D
                koff = (g * RPT + i) * K
                for c in range(NCHUNK):
                    iv = idxb[pl.ds(koff + c * 16, 16)]
                    pv = prb[pl.ds(koff + c * 16, 16)]
                    plsc.store_scatter(buf, [iv + roff], pv)
                return carry2

            lax.fori_loop(0, RPT, scatter_row, 0)
            pltpu.sync_copy(
                buf, out_hbm.at[pl.ds((base + g * RPT) * D, RPT * D)])

            def zero_row(i, carry2):
                roff = i * D
                koff = (g * RPT + i) * K
                for c in range(NCHUNK):
                    iv = idxb[pl.ds(koff + c * 16, 16)]
                    plsc.store_scatter(buf, [iv + roff], zv)
                return carry2

            lax.fori_loop(0, RPT, zero_row, 0)
            return carry

        lax.fori_loop(0, NG, group, 0)

    return sc, RPT


def kernel(moves_mem, idx, val):
    B, D = moves_mem.shape
    K = idx.shape[1]
    keys = (idx * B + jnp.arange(B, dtype=jnp.int32)[:, None]).reshape(-1)
    pos = jnp.arange(B * K, dtype=jnp.float32)
    sk, sp = lax.sort((keys, pos), dimension=0, is_stable=False, num_keys=1)
    flag = jnp.concatenate(
        [(sk[:-1] != sk[1:]).astype(jnp.int32),
         jnp.ones((1,), jnp.int32)])
    rk = _make_rank_scatter(B * K)
    rank = rk(sp.astype(jnp.int32), flag).reshape(B, K)
    probs = _compute_probs(idx, val, rank)
    sc, rpt = _make_scatter(B, D, K)
    zeros = jnp.zeros((rpt * D,), jnp.float32)
    out1d = sc(idx.reshape(-1), probs.reshape(-1), zeros)
    return out1d.reshape(B, D)
